# baseline (speedup 1.0000x reference)
import functools

import jax
import jax.numpy as jnp
from jax import lax
from jax.experimental import pallas as pl
from jax.experimental.pallas import tpu as pltpu
from jax.experimental.pallas import tpu_sc as plsc

B, L = 4096, 50
T = B * L
ED = 32
D = 4 * ED
IE_FILTER, HIDDEN = 256, 128
SHOW_ED = 8
POS_V = 200
CATE_V, PRICE_V = 1000, 100

NC, NS = 2, 16
NW = NC * NS
TPW = T // NW
CHUNK = 320
NCHUNKS = TPW // CHUNK
NCH2 = NCHUNKS // 2

BT = 1024


def _sc_gather(gi, si, ci, pi, gt, st, ct, pt):
    TL = gi.shape[0]
    tpw = TL // NW
    nch2 = tpw // CHUNK // 2
    mesh = plsc.VectorSubcoreMesh(core_axis_name="c", subcore_axis_name="s")

    @functools.partial(
        pl.kernel,
        out_type=jax.ShapeDtypeStruct((TL, D), jnp.float32),
        mesh=mesh,
        scratch_types=[
            [pltpu.VMEM((CHUNK,), jnp.int32) for _ in range(8)],
            [pltpu.VMEM((CHUNK, D), jnp.float32) for _ in range(2)],
            pltpu.SemaphoreType.DMA,
            pltpu.SemaphoreType.DMA,
            pltpu.SemaphoreType.DMA,
            pltpu.SemaphoreType.DMA,
        ],
    )
    def k(gi_h, si_h, ci_h, pi_h, gt_h, st_h, ct_h, pt_h, items_out,
          idx_v, items_v, sem_g, sem_i, sem_s0, sem_s1):
        wid = lax.axis_index("s") * NC + lax.axis_index("c")
        base = wid * tpw

        def load_idx(slot, c, sync):
            off = base + c * CHUNK
            for f, ids_h in enumerate((gi_h, si_h, ci_h, pi_h)):
                src = ids_h.at[pl.ds(off, CHUNK)]
                if sync:
                    pltpu.sync_copy(src, idx_v[slot * 4 + f])
                else:
                    pltpu.async_copy(src, idx_v[slot * 4 + f], sem_i)

        def gather_chunk(slot, c, sem_s):
            off = base + c * CHUNK
            buf = items_v[slot]
            ix = idx_v[slot * 4:slot * 4 + 4]
            pltpu.async_copy(gt_h.at[ix[0]], buf, sem_g).wait()
            d1 = pltpu.async_copy(st_h.at[ix[1]], buf, sem_g, add=True)
            d2 = pltpu.async_copy(ct_h.at[ix[2]], buf, sem_g, add=True)
            d3 = pltpu.async_copy(pt_h.at[ix[3]], buf, sem_g, add=True)
            d1.wait()
            d2.wait()
            d3.wait()
            pltpu.async_copy(buf, items_out.at[pl.ds(off, CHUNK)], sem_s)

        def drain_store(slot, sem_s):
            pltpu.make_async_copy(items_out.at[pl.ds(base, CHUNK)],
                                  items_v[slot], sem_s).wait()

        load_idx(0, 0, sync=True)
        load_idx(1, 1, sync=True)

        def body(m, carry):
            @pl.when(m > 0)
            def _():
                drain_store(0, sem_s0)
            gather_chunk(0, 2 * m, sem_s0)

            @pl.when(m < nch2 - 1)
            def _():
                load_idx(0, 2 * m + 2, sync=False)

            @pl.when(m > 0)
            def _():
                drain_store(1, sem_s1)
            gather_chunk(1, 2 * m + 1, sem_s1)

            @pl.when(m < nch2 - 1)
            def _():
                load_idx(1, 2 * m + 3, sync=False)
                for _ in range(8):
                    pltpu.make_async_copy(gi_h.at[pl.ds(base, CHUNK)],
                                          idx_v[0], sem_i).wait()
            return carry

        lax.fori_loop(0, nch2, body, 0)
        drain_store(0, sem_s0)
        drain_store(1, sem_s1)

    return k(gi, si, ci, pi, gt, st, ct, pt)


def _prep_band(table, k):
    V = table.shape[0]
    VB = 2048 if V > 2048 else V
    tt = jnp.swapaxes(table, 0, 1)

    def body(t_ref, o_ref):
        tr = t_ref[...].T
        z = jnp.zeros((VB, ED), jnp.float32)
        parts = [tr if j == k else z for j in range(4)]
        o_ref[...] = jnp.concatenate(parts, axis=1)

    return pl.pallas_call(
        body,
        grid=(pl.cdiv(V, VB),),
        in_specs=[pl.BlockSpec((ED, VB), lambda i: (0, i))],
        out_specs=pl.BlockSpec((VB, D), lambda i: (i, 0)),
        out_shape=jax.ShapeDtypeStruct((V, D), jnp.float32),
    )(tt)


def _tc_mlp(items4, rp_ids, sp_ids, rp_table, sp_table_t, w1, b1, w2, b2):
    TL = items4.shape[0]
    LL = TL // B
    NJ = B // BT

    def body(x_ref, rpi_ref, spi_ref, rpt_ref, spt_ref,
             w1_ref, b1_ref, w2_ref, b2_ref, out_ref, sp_ref):
        rpi = rpi_ref[0]
        spi = spi_ref[0]
        pos_iota = lax.broadcasted_iota(jnp.int32, (POS_V, BT), 0)
        oh_rp_t = (pos_iota == rpi).astype(jnp.bfloat16)
        oh_sp_t = (pos_iota == spi).astype(jnp.float32)
        rp = lax.dot_general(
            oh_rp_t, rpt_ref[...], (((0,), (0,)), ((), ())),
            preferred_element_type=jnp.float32)
        x = (x_ref[...] + rp).astype(jnp.bfloat16)
        sp_t = jnp.dot(spt_ref[...], oh_sp_t,
                       preferred_element_type=jnp.float32)
        sp_ref[...] = sp_t.reshape(1, SHOW_ED, BT)
        h = jnp.dot(x, w1_ref[...], preferred_element_type=jnp.float32)
        h = jnp.maximum(h + b1_ref[...], 0.0).astype(jnp.bfloat16)
        o = jnp.dot(h, w2_ref[...], preferred_element_type=jnp.float32)
        out_ref[...] = jnp.maximum(o + b2_ref[...], 0.0).reshape(1, BT, HIDDEN)

    grid = (TL // BT,)
    return pl.pallas_call(
        body,
        grid=grid,
        in_specs=[
            pl.BlockSpec((BT, D), lambda i: (i, 0)),
            pl.BlockSpec((1, 1, BT), lambda i: (i, 0, 0)),
            pl.BlockSpec((1, 1, BT), lambda i: (i, 0, 0)),
            pl.BlockSpec((POS_V, D), lambda i: (0, 0)),
            pl.BlockSpec((SHOW_ED, POS_V), lambda i: (0, 0)),
            pl.BlockSpec((D, IE_FILTER), lambda i: (0, 0)),
            pl.BlockSpec((1, IE_FILTER), lambda i: (0, 0)),
            pl.BlockSpec((IE_FILTER, HIDDEN), lambda i: (0, 0)),
            pl.BlockSpec((1, HIDDEN), lambda i: (0, 0)),
        ],
        out_specs=[
            pl.BlockSpec((1, BT, HIDDEN), lambda i: (i // NJ, i % NJ, 0)),
            pl.BlockSpec((1, SHOW_ED, BT), lambda i: (i // NJ, 0, i % NJ)),
        ],
        out_shape=[
            jax.ShapeDtypeStruct((LL, B, HIDDEN), jnp.float32),
            jax.ShapeDtypeStruct((LL, SHOW_ED, B), jnp.float32),
        ],
    )(items4, rp_ids.reshape(TL // BT, 1, BT), sp_ids.reshape(TL // BT, 1, BT),
      rp_table, sp_table_t, w1, b1.reshape(1, IE_FILTER), w2,
      b2.reshape(1, HIDDEN))


def kernel(goods_ids, shop_ids, cate_ids, gprice_ids, rankpos_ids, showpos_ids,
           goods_table, shop_table, cate_table, price_table, rankpos_table,
           showpos_table, gamma1, beta1, mean1, var1, W1, b1,
           gamma2, beta2, mean2, var2, W2, b2):
    gi, si, ci, pi, ri, wi = [
        jnp.swapaxes(a, 0, 1).reshape(T).astype(jnp.int32) for a in
        (goods_ids, shop_ids, cate_ids, gprice_ids, rankpos_ids, showpos_ids)]

    gt = _prep_band(goods_table, 0)
    st = _prep_band(shop_table, 1)
    ct = _prep_band(cate_table, 2)
    pt = _prep_band(price_table, 3)

    eps = 1e-6
    a1 = gamma1 * lax.rsqrt(var1 + eps)
    c1 = beta1 - mean1 * a1
    w1 = W1 * a1[:, None]
    b1f = b1 + c1 @ W1
    a2 = gamma2 * lax.rsqrt(var2 + eps)
    c2 = beta2 - mean2 * a2
    w2 = W2 * a2[:, None]
    b2f = b2 + c2 @ W2

    T2 = T // 2
    rpt_b = rankpos_table.astype(jnp.bfloat16)
    spt_t = jnp.swapaxes(showpos_table, 0, 1)
    w1b = w1.astype(jnp.bfloat16)
    w2b = w2.astype(jnp.bfloat16)
    halves = []
    for s in (slice(0, T2), slice(T2, T)):
        items4 = _sc_gather(gi[s], si[s], ci[s], pi[s], gt, st, ct, pt)
        halves.append(_tc_mlp(items4, ri[s], wi[s], rpt_b, spt_t,
                              w1b, b1f, w2b, b2f))

    out_lb = jnp.concatenate([halves[0][0], halves[1][0]], axis=0)
    sp_lb = jnp.concatenate([halves[0][1], halves[1][1]], axis=0)
    sequence_len = jnp.full((B,), L, dtype=jnp.int32)
    return (jnp.transpose(out_lb, (1, 0, 2)), sequence_len,
            jnp.transpose(sp_lb, (2, 0, 1)))


# speedup vs baseline: 1.1207x; 1.1207x over previous
import functools

import jax
import jax.numpy as jnp
from jax import lax
from jax.experimental import pallas as pl
from jax.experimental.pallas import tpu as pltpu
from jax.experimental.pallas import tpu_sc as plsc

B, L = 4096, 50
T = B * L
ED = 32
D = 4 * ED
IE_FILTER, HIDDEN = 256, 128
SHOW_ED = 8
POS_V = 200
CATE_V, PRICE_V = 1000, 100

NC, NS = 2, 16
NW = NC * NS
TPW = T // NW
CHUNK = 320
NCHUNKS = TPW // CHUNK
NCH2 = NCHUNKS // 2

BT = 2048


def _sc_gather(gi, si, ci, pi, gt, st, ct, pt):
    mesh = plsc.VectorSubcoreMesh(core_axis_name="c", subcore_axis_name="s")

    @functools.partial(
        pl.kernel,
        out_type=jax.ShapeDtypeStruct((T, D), jnp.float32),
        mesh=mesh,
        scratch_types=[
            [pltpu.VMEM((CHUNK,), jnp.int32) for _ in range(8)],
            [pltpu.VMEM((CHUNK, D), jnp.float32) for _ in range(2)],
            pltpu.SemaphoreType.DMA,
            pltpu.SemaphoreType.DMA,
            pltpu.SemaphoreType.DMA,
            pltpu.SemaphoreType.DMA,
        ],
    )
    def k(gi_h, si_h, ci_h, pi_h, gt_h, st_h, ct_h, pt_h, items_out,
          idx_v, items_v, sem_g, sem_i, sem_s0, sem_s1):
        wid = lax.axis_index("s") * NC + lax.axis_index("c")
        base = wid * TPW

        def load_idx(slot, c, sync):
            off = base + c * CHUNK
            for f, ids_h in enumerate((gi_h, si_h, ci_h, pi_h)):
                src = ids_h.at[pl.ds(off, CHUNK)]
                if sync:
                    pltpu.sync_copy(src, idx_v[slot * 4 + f])
                else:
                    pltpu.async_copy(src, idx_v[slot * 4 + f], sem_i)

        def gather_chunk(slot, c, sem_s):
            off = base + c * CHUNK
            buf = items_v[slot]
            ix = idx_v[slot * 4:slot * 4 + 4]
            pltpu.async_copy(gt_h.at[ix[0]], buf, sem_g).wait()
            d1 = pltpu.async_copy(st_h.at[ix[1]], buf, sem_g, add=True)
            d2 = pltpu.async_copy(ct_h.at[ix[2]], buf, sem_g, add=True)
            d3 = pltpu.async_copy(pt_h.at[ix[3]], buf, sem_g, add=True)
            d1.wait()
            d2.wait()
            d3.wait()
            pltpu.async_copy(buf, items_out.at[pl.ds(off, CHUNK)], sem_s)

        def drain_store(slot, sem_s):
            pltpu.make_async_copy(items_out.at[pl.ds(base, CHUNK)],
                                  items_v[slot], sem_s).wait()

        load_idx(0, 0, sync=True)
        load_idx(1, 1, sync=True)

        def body(m, carry):
            @pl.when(m > 0)
            def _():
                drain_store(0, sem_s0)
            gather_chunk(0, 2 * m, sem_s0)

            @pl.when(m < NCH2 - 1)
            def _():
                load_idx(0, 2 * m + 2, sync=False)

            @pl.when(m > 0)
            def _():
                drain_store(1, sem_s1)
            gather_chunk(1, 2 * m + 1, sem_s1)

            @pl.when(m < NCH2 - 1)
            def _():
                load_idx(1, 2 * m + 3, sync=False)
                for _ in range(8):
                    pltpu.make_async_copy(gi_h.at[pl.ds(base, CHUNK)],
                                          idx_v[0], sem_i).wait()
            return carry

        lax.fori_loop(0, NCH2, body, 0)
        drain_store(0, sem_s0)
        drain_store(1, sem_s1)

    return k(gi, si, ci, pi, gt, st, ct, pt)


def _prep_band(table, k):
    V = table.shape[0]
    VB = 2048 if V > 2048 else V
    tt = jnp.swapaxes(table, 0, 1)

    def body(t_ref, o_ref):
        tr = t_ref[...].T
        z = jnp.zeros((VB, ED), jnp.float32)
        parts = [tr if j == k else z for j in range(4)]
        o_ref[...] = jnp.concatenate(parts, axis=1)

    return pl.pallas_call(
        body,
        grid=(pl.cdiv(V, VB),),
        in_specs=[pl.BlockSpec((ED, VB), lambda i: (0, i))],
        out_specs=pl.BlockSpec((VB, D), lambda i: (i, 0)),
        out_shape=jax.ShapeDtypeStruct((V, D), jnp.float32),
    )(tt)


def _tc_mlp(items4, rp_ids, sp_ids, rp_table, sp_table_t, w1, b1, w2, b2):
    NJ = B // BT

    def body(x_ref, rpi_ref, spi_ref, rpt_ref, spt_ref,
             w1_ref, b1_ref, w2_ref, b2_ref, out_ref, sp_ref):
        rpi = rpi_ref[0]
        spi = spi_ref[0]
        pos_iota = lax.broadcasted_iota(jnp.int32, (POS_V, BT), 0)
        oh_rp_t = (pos_iota == rpi).astype(jnp.bfloat16)
        oh_sp_t = (pos_iota == spi).astype(jnp.float32)
        rp = lax.dot_general(
            oh_rp_t, rpt_ref[...], (((0,), (0,)), ((), ())),
            preferred_element_type=jnp.float32)
        x = (x_ref[...] + rp).astype(jnp.bfloat16)
        sp_t = jnp.dot(spt_ref[...], oh_sp_t,
                       preferred_element_type=jnp.float32)
        sp_ref[...] = sp_t.reshape(1, SHOW_ED, BT)
        h = jnp.dot(x, w1_ref[...], preferred_element_type=jnp.float32)
        h = jnp.maximum(h + b1_ref[...], 0.0).astype(jnp.bfloat16)
        o = jnp.dot(h, w2_ref[...], preferred_element_type=jnp.float32)
        out_ref[...] = jnp.maximum(o + b2_ref[...], 0.0).reshape(1, BT, HIDDEN)

    grid = (T // BT,)
    return pl.pallas_call(
        body,
        grid=grid,
        in_specs=[
            pl.BlockSpec((BT, D), lambda i: (i, 0)),
            pl.BlockSpec((1, 1, BT), lambda i: (i, 0, 0)),
            pl.BlockSpec((1, 1, BT), lambda i: (i, 0, 0)),
            pl.BlockSpec((POS_V, D), lambda i: (0, 0)),
            pl.BlockSpec((SHOW_ED, POS_V), lambda i: (0, 0)),
            pl.BlockSpec((D, IE_FILTER), lambda i: (0, 0)),
            pl.BlockSpec((1, IE_FILTER), lambda i: (0, 0)),
            pl.BlockSpec((IE_FILTER, HIDDEN), lambda i: (0, 0)),
            pl.BlockSpec((1, HIDDEN), lambda i: (0, 0)),
        ],
        out_specs=[
            pl.BlockSpec((1, BT, HIDDEN), lambda i: (i // NJ, i % NJ, 0)),
            pl.BlockSpec((1, SHOW_ED, BT), lambda i: (i // NJ, 0, i % NJ)),
        ],
        out_shape=[
            jax.ShapeDtypeStruct((L, B, HIDDEN), jnp.float32),
            jax.ShapeDtypeStruct((L, SHOW_ED, B), jnp.float32),
        ],
    )(items4, rp_ids.reshape(T // BT, 1, BT), sp_ids.reshape(T // BT, 1, BT),
      rp_table, sp_table_t, w1, b1.reshape(1, IE_FILTER), w2,
      b2.reshape(1, HIDDEN))


def kernel(goods_ids, shop_ids, cate_ids, gprice_ids, rankpos_ids, showpos_ids,
           goods_table, shop_table, cate_table, price_table, rankpos_table,
           showpos_table, gamma1, beta1, mean1, var1, W1, b1,
           gamma2, beta2, mean2, var2, W2, b2):
    gi, si, ci, pi, ri, wi = [
        jnp.swapaxes(a, 0, 1).reshape(T).astype(jnp.int32) for a in
        (goods_ids, shop_ids, cate_ids, gprice_ids, rankpos_ids, showpos_ids)]

    gt = _prep_band(goods_table, 0)
    st = _prep_band(shop_table, 1)
    ct = _prep_band(cate_table, 2)
    pt = _prep_band(price_table, 3)

    eps = 1e-6
    a1 = gamma1 * lax.rsqrt(var1 + eps)
    c1 = beta1 - mean1 * a1
    w1 = W1 * a1[:, None]
    b1f = b1 + c1 @ W1
    a2 = gamma2 * lax.rsqrt(var2 + eps)
    c2 = beta2 - mean2 * a2
    w2 = W2 * a2[:, None]
    b2f = b2 + c2 @ W2

    items4 = _sc_gather(gi, si, ci, pi, gt, st, ct, pt)
    out_lb, sp_lb = _tc_mlp(items4, ri, wi,
                            rankpos_table.astype(jnp.bfloat16),
                            jnp.swapaxes(showpos_table, 0, 1),
                            w1.astype(jnp.bfloat16), b1f,
                            w2.astype(jnp.bfloat16), b2f)

    sequence_len = jnp.full((B,), L, dtype=jnp.int32)
    return (jnp.transpose(out_lb, (1, 0, 2)), sequence_len,
            jnp.transpose(sp_lb, (2, 0, 1)))


# speedup vs baseline: 1.1785x; 1.0516x over previous
import functools

import jax
import jax.numpy as jnp
from jax import lax
from jax.experimental import pallas as pl
from jax.experimental.pallas import tpu as pltpu
from jax.experimental.pallas import tpu_sc as plsc

B, L = 4096, 50
T = B * L
ED = 32
D = 4 * ED
IE_FILTER, HIDDEN = 256, 128
SHOW_ED = 8
POS_V = 200
CATE_V, PRICE_V = 1000, 100

NC, NS = 2, 16
NW = NC * NS
TPW = T // NW
CHUNK = 320
NCHUNKS = TPW // CHUNK
NCH2 = NCHUNKS // 2

BT = 4096


def _sc_gather(gi, si, ci, pi, gt, st, ct, pt):
    mesh = plsc.VectorSubcoreMesh(core_axis_name="c", subcore_axis_name="s")

    @functools.partial(
        pl.kernel,
        out_type=jax.ShapeDtypeStruct((T, D), jnp.float32),
        mesh=mesh,
        scratch_types=[
            [pltpu.VMEM((CHUNK,), jnp.int32) for _ in range(8)],
            [pltpu.VMEM((CHUNK, D), jnp.float32) for _ in range(2)],
            pltpu.SemaphoreType.DMA,
            pltpu.SemaphoreType.DMA,
            pltpu.SemaphoreType.DMA,
            pltpu.SemaphoreType.DMA,
        ],
    )
    def k(gi_h, si_h, ci_h, pi_h, gt_h, st_h, ct_h, pt_h, items_out,
          idx_v, items_v, sem_g, sem_i, sem_s0, sem_s1):
        wid = lax.axis_index("s") * NC + lax.axis_index("c")
        base = wid * TPW

        def load_idx(slot, c, sync):
            off = base + c * CHUNK
            for f, ids_h in enumerate((gi_h, si_h, ci_h, pi_h)):
                src = ids_h.at[pl.ds(off, CHUNK)]
                if sync:
                    pltpu.sync_copy(src, idx_v[slot * 4 + f])
                else:
                    pltpu.async_copy(src, idx_v[slot * 4 + f], sem_i)

        def gather_chunk(slot, c, sem_s):
            off = base + c * CHUNK
            buf = items_v[slot]
            ix = idx_v[slot * 4:slot * 4 + 4]
            pltpu.async_copy(gt_h.at[ix[0]], buf, sem_g).wait()
            d1 = pltpu.async_copy(st_h.at[ix[1]], buf, sem_g, add=True)
            d2 = pltpu.async_copy(ct_h.at[ix[2]], buf, sem_g, add=True)
            d3 = pltpu.async_copy(pt_h.at[ix[3]], buf, sem_g, add=True)
            d1.wait()
            d2.wait()
            d3.wait()
            pltpu.async_copy(buf, items_out.at[pl.ds(off, CHUNK)], sem_s)

        def drain_store(slot, sem_s):
            pltpu.make_async_copy(items_out.at[pl.ds(base, CHUNK)],
                                  items_v[slot], sem_s).wait()

        load_idx(0, 0, sync=True)
        load_idx(1, 1, sync=True)

        def body(m, carry):
            @pl.when(m > 0)
            def _():
                drain_store(0, sem_s0)
            gather_chunk(0, 2 * m, sem_s0)

            @pl.when(m < NCH2 - 1)
            def _():
                load_idx(0, 2 * m + 2, sync=False)

            @pl.when(m > 0)
            def _():
                drain_store(1, sem_s1)
            gather_chunk(1, 2 * m + 1, sem_s1)

            @pl.when(m < NCH2 - 1)
            def _():
                load_idx(1, 2 * m + 3, sync=False)
                for _ in range(8):
                    pltpu.make_async_copy(gi_h.at[pl.ds(base, CHUNK)],
                                          idx_v[0], sem_i).wait()
            return carry

        lax.fori_loop(0, NCH2, body, 0)
        drain_store(0, sem_s0)
        drain_store(1, sem_s1)

    return k(gi, si, ci, pi, gt, st, ct, pt)


def _prep_band(table, k):
    V = table.shape[0]
    VB = 2048 if V > 2048 else V
    tt = jnp.swapaxes(table, 0, 1)

    def body(t_ref, o_ref):
        tr = t_ref[...].T
        z = jnp.zeros((VB, ED), jnp.float32)
        parts = [tr if j == k else z for j in range(4)]
        o_ref[...] = jnp.concatenate(parts, axis=1)

    return pl.pallas_call(
        body,
        grid=(pl.cdiv(V, VB),),
        in_specs=[pl.BlockSpec((ED, VB), lambda i: (0, i))],
        out_specs=pl.BlockSpec((VB, D), lambda i: (i, 0)),
        out_shape=jax.ShapeDtypeStruct((V, D), jnp.float32),
    )(tt)


def _tc_mlp(items4, rp_ids, sp_ids, rp_table, sp_table_t, w1, b1, w2, b2):
    NJ = B // BT

    def body(x_ref, rpi_ref, spi_ref, rpt_ref, spt_ref,
             w1_ref, b1_ref, w2_ref, b2_ref, out_ref, sp_ref):
        rpi = rpi_ref[0]
        spi = spi_ref[0]
        pos_iota = lax.broadcasted_iota(jnp.int32, (POS_V, BT), 0)
        oh_rp_t = (pos_iota == rpi).astype(jnp.bfloat16)
        oh_sp_t = (pos_iota == spi).astype(jnp.float32)
        rp = lax.dot_general(
            oh_rp_t, rpt_ref[...], (((0,), (0,)), ((), ())),
            preferred_element_type=jnp.float32)
        x = (x_ref[...] + rp).astype(jnp.bfloat16)
        sp_t = jnp.dot(spt_ref[...], oh_sp_t,
                       preferred_element_type=jnp.float32)
        sp_ref[...] = sp_t.reshape(1, SHOW_ED, BT)
        h = jnp.dot(x, w1_ref[...], preferred_element_type=jnp.float32)
        h = jnp.maximum(h + b1_ref[...], 0.0).astype(jnp.bfloat16)
        o = jnp.dot(h, w2_ref[...], preferred_element_type=jnp.float32)
        out_ref[...] = jnp.maximum(o + b2_ref[...], 0.0).reshape(1, BT, HIDDEN)

    grid = (T // BT,)
    return pl.pallas_call(
        body,
        grid=grid,
        in_specs=[
            pl.BlockSpec((BT, D), lambda i: (i, 0)),
            pl.BlockSpec((1, 1, BT), lambda i: (i, 0, 0)),
            pl.BlockSpec((1, 1, BT), lambda i: (i, 0, 0)),
            pl.BlockSpec((POS_V, D), lambda i: (0, 0)),
            pl.BlockSpec((SHOW_ED, POS_V), lambda i: (0, 0)),
            pl.BlockSpec((D, IE_FILTER), lambda i: (0, 0)),
            pl.BlockSpec((1, IE_FILTER), lambda i: (0, 0)),
            pl.BlockSpec((IE_FILTER, HIDDEN), lambda i: (0, 0)),
            pl.BlockSpec((1, HIDDEN), lambda i: (0, 0)),
        ],
        out_specs=[
            pl.BlockSpec((1, BT, HIDDEN), lambda i: (i // NJ, i % NJ, 0)),
            pl.BlockSpec((1, SHOW_ED, BT), lambda i: (i // NJ, 0, i % NJ)),
        ],
        out_shape=[
            jax.ShapeDtypeStruct((L, B, HIDDEN), jnp.float32),
            jax.ShapeDtypeStruct((L, SHOW_ED, B), jnp.float32),
        ],
    )(items4, rp_ids.reshape(T // BT, 1, BT), sp_ids.reshape(T // BT, 1, BT),
      rp_table, sp_table_t, w1, b1.reshape(1, IE_FILTER), w2,
      b2.reshape(1, HIDDEN))


def kernel(goods_ids, shop_ids, cate_ids, gprice_ids, rankpos_ids, showpos_ids,
           goods_table, shop_table, cate_table, price_table, rankpos_table,
           showpos_table, gamma1, beta1, mean1, var1, W1, b1,
           gamma2, beta2, mean2, var2, W2, b2):
    gi, si, ci, pi, ri, wi = [
        jnp.swapaxes(a, 0, 1).reshape(T).astype(jnp.int32) for a in
        (goods_ids, shop_ids, cate_ids, gprice_ids, rankpos_ids, showpos_ids)]

    gt = _prep_band(goods_table, 0)
    st = _prep_band(shop_table, 1)
    ct = _prep_band(cate_table, 2)
    pt = _prep_band(price_table, 3)

    eps = 1e-6
    a1 = gamma1 * lax.rsqrt(var1 + eps)
    c1 = beta1 - mean1 * a1
    w1 = W1 * a1[:, None]
    b1f = b1 + c1 @ W1
    a2 = gamma2 * lax.rsqrt(var2 + eps)
    c2 = beta2 - mean2 * a2
    w2 = W2 * a2[:, None]
    b2f = b2 + c2 @ W2

    items4 = _sc_gather(gi, si, ci, pi, gt, st, ct, pt)
    out_lb, sp_lb = _tc_mlp(items4, ri, wi,
                            rankpos_table.astype(jnp.bfloat16),
                            jnp.swapaxes(showpos_table, 0, 1),
                            w1.astype(jnp.bfloat16), b1f,
                            w2.astype(jnp.bfloat16), b2f)

    sequence_len = jnp.full((B,), L, dtype=jnp.int32)
    return (jnp.transpose(out_lb, (1, 0, 2)), sequence_len,
            jnp.transpose(sp_lb, (2, 0, 1)))
